# baseline (speedup 1.0000x reference)
import functools

import jax
import jax.numpy as jnp
from jax import lax
from jax.experimental import pallas as pl
from jax.experimental.pallas import tpu as pltpu
from jax.experimental.pallas import tpu_sc as plsc

_CHUNK = 128
_L = 16
_CHR = 160
_OB = 128


@functools.cache
def _build_gather(B: int, V: int, D: int):
    info = plsc.get_sparse_core_info()
    NC, NS = info.num_cores, info.num_subcores
    NW = NC * NS
    assert B % (NW * _CHUNK) == 0
    assert D == 4
    b_per_w = B // NW
    e_per_w = b_per_w * D
    r_stride = (V // NS) & ~7
    r_per_t = V - (NS - 1) * r_stride
    assert r_per_t >= r_stride and r_per_t % 8 == 0
    chunks = [_CHR] * (r_per_t // _CHR)
    if r_per_t % _CHR:
        chunks.append(r_per_t % _CHR)
    assert all(c % 8 == 0 for c in chunks)
    mesh = plsc.VectorSubcoreMesh(core_axis_name="c", subcore_axis_name="s")

    @functools.partial(
        pl.kernel,
        mesh=mesh,
        out_type=jax.ShapeDtypeStruct((B, D), jnp.float32),
        compiler_params=pltpu.CompilerParams(
            use_tc_tiling_on_sc=True, needs_layout_passes=False
        ),
        scratch_types=[
            pltpu.HBM((V * D,), jnp.float32),
            pltpu.VMEM((_CHR, D), jnp.float32),
            pltpu.VMEM((_CHR, D), jnp.float32),
            pltpu.VMEM((_CHR * D,), jnp.float32),
            pltpu.VMEM((_CHR * D,), jnp.float32),
            pltpu.VMEM((b_per_w,), jnp.int32),
            pltpu.VMEM((e_per_w,), jnp.int32),
            pltpu.VMEM((e_per_w,), jnp.float32),
            pltpu.VMEM((_OB, D), jnp.float32),
            pltpu.SemaphoreType.DMA,
            pltpu.SemaphoreType.DMA,
            pltpu.SemaphoreType.DMA,
            pltpu.SemaphoreType.DMA,
            pltpu.SemaphoreType.DMA,
        ],
    )
    def gather_k(ids_hbm, table_hbm, out_hbm, flat_hbm, bufA, bufB, fbA, fbB,
                 idx_v, eidx_v, valsf_v, vals2_v, semA, semB, semFA, semFB,
                 gsem):
        cid = lax.axis_index("c")
        sid = lax.axis_index("s")
        wid = sid * NC + cid
        lanes = lax.iota(jnp.int32, _L)
        base = wid * b_per_w
        r0 = sid * r_stride

        bufs = [(bufA, fbA, semA, semFA), (bufB, fbB, semB, semFB)]
        offs = [sum(chunks[:i]) for i in range(len(chunks))]

        def issue_stage(k):
            buf, _, sem, _ = bufs[k % 2]
            return pltpu.async_copy(
                table_hbm.at[pl.ds(r0 + offs[k], chunks[k])],
                buf.at[pl.ds(0, chunks[k])],
                sem,
            )

        stage0 = issue_stage(0)

        pltpu.sync_copy(ids_hbm.at[pl.ds(base, b_per_w)], idx_v)
        for k in range(b_per_w // _L):
            v4 = idx_v[pl.ds(k * _L, _L)] * D
            pos = lanes * D + (k * _L * D)
            for c in range(D):
                plsc.store_scatter(eidx_v, [pos + c], v4 + c)

        stage_h = stage0
        flat_h = [None, None]
        for k in range(len(chunks)):
            buf, fb, _, fsem = bufs[k % 2]
            stage_h.wait()
            if k + 1 < len(chunks):
                next_h = issue_stage(k + 1)
            w_k = chunks[k] * D

            def repack_body(j, carry, buf=buf, fb=fb):
                e = lanes + j * _L
                v = plsc.load_gather(buf, [e >> 2, e & 3])
                plsc.store_scatter(fb, [e], v)
                return carry

            if flat_h[k % 2] is not None:
                flat_h[k % 2].wait()
            lax.fori_loop(0, w_k // _L, repack_body, 0, unroll=4)
            flat_h[k % 2] = pltpu.async_copy(
                fb.at[pl.ds(0, w_k)],
                flat_hbm.at[pl.ds((r0 + offs[k]) * D, w_k)],
                fsem,
            )
            if k + 1 < len(chunks):
                stage_h = next_h
        for h in flat_h:
            if h is not None:
                h.wait()
        plsc.subcore_barrier()

        copies = [
            pltpu.async_copy(
                flat_hbm.at[eidx_v.at[pl.ds(j * _CHUNK, _CHUNK)]],
                valsf_v.at[pl.ds(j * _CHUNK, _CHUNK)],
                gsem,
            )
            for j in range(e_per_w // _CHUNK)
        ]
        for cpy in copies:
            cpy.wait()

        for k2 in range(b_per_w // _OB):
            e0 = k2 * _OB * D

            def out_repack(j, carry, e0=e0):
                e = lanes + j * _L
                v = valsf_v[pl.ds(e0 + j * _L, _L)]
                plsc.store_scatter(vals2_v, [e >> 2, e & 3], v)
                return carry

            lax.fori_loop(0, _OB * D // _L, out_repack, 0, unroll=4)
            pltpu.sync_copy(
                vals2_v, out_hbm.at[pl.ds(base + k2 * _OB, _OB)]
            )

    return gather_k


def kernel(ids, data):
    B, = ids.shape
    V, D = data.shape
    gather_k = _build_gather(B, V, D)
    return gather_k(ids.astype(jnp.int32), data)


# speedup vs baseline: 3.0812x; 3.0812x over previous
import functools

import jax
import jax.numpy as jnp
from jax import lax
from jax.experimental import pallas as pl
from jax.experimental.pallas import tpu as pltpu
from jax.experimental.pallas import tpu_sc as plsc

_CHUNK = 128
_L = 16
_OB = 128


@functools.cache
def _build_gather(B: int, V: int, D: int):
    info = plsc.get_sparse_core_info()
    NC, NS = info.num_cores, info.num_subcores
    NW = NC * NS
    assert B % (NW * _CHUNK) == 0
    assert D == 4
    b_per_w = B // NW
    mesh = plsc.VectorSubcoreMesh(core_axis_name="c", subcore_axis_name="s")

    @functools.partial(
        pl.kernel,
        mesh=mesh,
        out_type=jax.ShapeDtypeStruct((B, D), jnp.float32),
        compiler_params=pltpu.CompilerParams(
            use_tc_tiling_on_sc=False, needs_layout_passes=False
        ),
        scratch_types=[
            pltpu.VMEM((b_per_w,), jnp.int32),
            pltpu.VMEM((D, b_per_w), jnp.float32),
            pltpu.VMEM((_OB, D), jnp.float32),
            pltpu.SemaphoreType.DMA,
        ],
    )
    def gather_k(ids_hbm, dataT_hbm, out_hbm, idx_v, colv, vals2, sem):
        wid = lax.axis_index("s") * NC + lax.axis_index("c")
        base = wid * b_per_w
        lanes = lax.iota(jnp.int32, _L)

        pltpu.sync_copy(ids_hbm.at[pl.ds(base, b_per_w)], idx_v)
        copies = []
        for c in range(D):
            col = dataT_hbm.at[c]
            for j in range(b_per_w // _CHUNK):
                copies.append(pltpu.async_copy(
                    col.at[idx_v.at[pl.ds(j * _CHUNK, _CHUNK)]],
                    colv.at[c, pl.ds(j * _CHUNK, _CHUNK)],
                    sem,
                ))
        for cp in copies:
            cp.wait()

        for k2 in range(b_per_w // _OB):
            def interleave(j, carry, k2=k2):
                e = lanes + j * _L
                v = plsc.load_gather(colv, [e & 3, (e >> 2) + k2 * _OB])
                plsc.store_scatter(vals2, [e >> 2, e & 3], v)
                return carry

            lax.fori_loop(0, _OB * D // _L, interleave, 0, unroll=4)
            pltpu.sync_copy(vals2, out_hbm.at[pl.ds(base + k2 * _OB, _OB)])

    return gather_k


def kernel(ids, data):
    B, = ids.shape
    V, D = data.shape
    gather_k = _build_gather(B, V, D)
    return gather_k(ids.astype(jnp.int32), data.T)


# speedup vs baseline: 4.5029x; 1.4614x over previous
import functools

import jax
import jax.numpy as jnp
from jax import lax
from jax.experimental import pallas as pl
from jax.experimental.pallas import tpu as pltpu
from jax.experimental.pallas import tpu_sc as plsc

_CHUNK = 128


@functools.cache
def _build_gather(B: int, V: int, D: int):
    info = plsc.get_sparse_core_info()
    NC, NS = info.num_cores, info.num_subcores
    NW = NC * NS
    assert B % (NW * _CHUNK) == 0
    b_per_w = B // NW
    mesh = plsc.VectorSubcoreMesh(core_axis_name="c", subcore_axis_name="s")

    @functools.partial(
        pl.kernel,
        mesh=mesh,
        out_type=jax.ShapeDtypeStruct((D, B), jnp.float32),
        compiler_params=pltpu.CompilerParams(
            use_tc_tiling_on_sc=False, needs_layout_passes=False
        ),
        scratch_types=[
            pltpu.VMEM((b_per_w,), jnp.int32),
            pltpu.VMEM((D, b_per_w), jnp.float32),
            pltpu.SemaphoreType.DMA,
        ],
    )
    def gather_k(ids_hbm, dataT_hbm, outT_hbm, idx_v, colv, sem):
        wid = lax.axis_index("s") * NC + lax.axis_index("c")
        base = wid * b_per_w

        pltpu.sync_copy(ids_hbm.at[pl.ds(base, b_per_w)], idx_v)
        copies = []
        for c in range(D):
            col = dataT_hbm.at[c]
            for j in range(b_per_w // _CHUNK):
                copies.append(pltpu.async_copy(
                    col.at[idx_v.at[pl.ds(j * _CHUNK, _CHUNK)]],
                    colv.at[c, pl.ds(j * _CHUNK, _CHUNK)],
                    sem,
                ))
        for cp in copies:
            cp.wait()
        for c in range(D):
            pltpu.sync_copy(
                colv.at[c], outT_hbm.at[c, pl.ds(base, b_per_w)]
            )

    return gather_k


def kernel(ids, data):
    B, = ids.shape
    V, D = data.shape
    gather_k = _build_gather(B, V, D)
    return gather_k(ids.astype(jnp.int32), data.T).T
